# baseline (speedup 1.0000x reference)
import functools

import jax
import jax.numpy as jnp
from jax import lax
from jax.experimental import pallas as pl
from jax.experimental.pallas import tpu as pltpu
from jax.experimental.pallas import tpu_sc as plsc

N = 1000
NP = 1024
D = 128
HID = 512
T_STEPS = 10
E_REAL = 17000
E_ROWS = 136
E_PAD = E_ROWS * 128
SENTINEL = 1023 * NP + 1023
RPT = NP // 32

E_B = 2000
K_TILES = 16000 // E_B


def _sc_counts(flat2d):
    mesh = plsc.VectorSubcoreMesh(core_axis_name="c", subcore_axis_name="s")

    @functools.partial(
        pl.kernel,
        out_type=jax.ShapeDtypeStruct((8 * NP, 128), jnp.float32),
        mesh=mesh,
        compiler_params=pltpu.CompilerParams(needs_layout_passes=False),
        scratch_types=[
            pltpu.VMEM((E_ROWS, 128), jnp.int32),
            pltpu.VMEM((RPT * 8 * 128,), jnp.float32),
            pltpu.VMEM((RPT * 8, 128), jnp.float32),
        ],
    )
    def sc_kernel(flat_hbm, out_hbm, idx_v, cflat_v, c2d_v):
        c = lax.axis_index("c")
        s = lax.axis_index("s")
        w = s * 2 + c
        lo = w * RPT
        pltpu.sync_copy(flat_hbm, idx_v)

        zero16 = jnp.zeros((16,), jnp.float32)
        ones16 = jnp.full((16,), 1.0, jnp.float32)

        def zz(i, _):
            cflat_v[pl.ds(i * 16, 16)] = zero16
            return 0

        lax.fori_loop(0, RPT * 8 * 8, zz, 0)

        def scan(gi, _):
            fl = idx_v[gi // 8, pl.ds((gi % 8) * 16, 16)]
            d = lax.shift_right_logical(fl, 10)
            lane = lax.bitwise_and(fl, 127)
            j = lax.bitwise_and(lax.shift_right_logical(fl, 7), 7)
            dl = d - lo
            m = (dl >= 0) & (dl < RPT)
            tgt = (j * RPT + dl) * 128 + lane
            tgt = jnp.where(m, tgt, 0)
            plsc.addupdate_scatter(cflat_v, [tgt], ones16, mask=m)
            return 0

        lax.fori_loop(0, E_PAD // 16, scan, 0)

        def cp(r, _):
            for g in range(8):
                c2d_v[r, pl.ds(g * 16, 16)] = cflat_v[pl.ds(r * 128 + g * 16, 16)]
            return 0

        lax.fori_loop(0, RPT * 8, cp, 0)

        for j in range(8):
            pltpu.sync_copy(c2d_v.at[pl.ds(j * RPT, RPT)],
                            out_hbm.at[pl.ds(j * NP + w * RPT, RPT)])

    return sc_kernel(flat2d)


def _dot16(a, b):
    return jnp.dot(a.astype(jnp.bfloat16), b.astype(jnp.bfloat16),
                   preferred_element_type=jnp.float32)


def _tc_body(x_ref, craw_ref, min_ref, mout_ref, wxt_ref, wht_ref, b4_ref,
             wm1t_ref, wm2t_ref, bm_ref, wct_ref, bc_ref, out_ref,
             a16_ref, aproj_ref, bproj_ref, xn_ref):
    k = pl.program_id(0)

    @pl.when(k == 0)
    def _prologue():
        cs = jnp.concatenate(
            [craw_ref[j * NP:j * NP + N, :] for j in range(8)], axis=1)[:, :N]
        deg = jnp.sum(cs, axis=1)
        dinv = lax.rsqrt(deg)
        a16_ref[...] = (cs * dinv[:, None] * dinv[None, :]).astype(jnp.bfloat16)

        wx16 = wxt_ref[...].astype(jnp.bfloat16)
        wh16 = wht_ref[...].astype(jnp.bfloat16)

        def step(t, hc):
            h, cc = hc
            xt = x_ref[t].astype(jnp.bfloat16)
            xw = (jnp.dot(xt, wx16, preferred_element_type=jnp.float32)
                  + jnp.dot(h.astype(jnp.bfloat16), wh16,
                            preferred_element_type=jnp.float32))
            z = (jnp.dot(a16_ref[...], xw.astype(jnp.bfloat16),
                         preferred_element_type=jnp.float32)
                 + b4_ref[...])
            ig = jax.nn.sigmoid(z[:, 0:128])
            fg = jax.nn.sigmoid(z[:, 128:256])
            og = jax.nn.sigmoid(z[:, 256:384])
            gg = jnp.tanh(z[:, 384:512])
            cc = fg * cc + ig * gg
            h = og * jnp.tanh(cc)
            return h, cc

        h0 = jnp.zeros((N, D), jnp.float32)
        h, _ = lax.fori_loop(0, T_STEPS, step, (h0, h0))
        aproj_ref[...] = _dot16(h, wm1t_ref[...]).astype(jnp.bfloat16)
        bproj_ref[...] = _dot16(h, wm2t_ref[...]).astype(jnp.bfloat16)
        xn_ref[...] = jnp.zeros((D, N), jnp.float32)

    mi16 = min_ref[...]
    e = jnp.maximum(
        jnp.dot(mi16, aproj_ref[...], preferred_element_type=jnp.float32)
        + jnp.dot(mout_ref[...], bproj_ref[...],
                  preferred_element_type=jnp.float32)
        + bm_ref[...], 0.0)
    xn_ref[...] += lax.dot_general(e.astype(jnp.bfloat16), mi16,
                                   (((0,), (0,)), ((), ())),
                                   preferred_element_type=jnp.float32)

    @pl.when(k == K_TILES - 1)
    def _epilogue():
        xw = lax.dot_general(
            (xn_ref[...] * (1.0 / N)).astype(jnp.bfloat16),
            wct_ref[...].astype(jnp.bfloat16),
            (((0,), (0,)), ((), ())),
            preferred_element_type=jnp.float32)
        out_ref[...] = (jnp.dot(a16_ref[...], xw.astype(jnp.bfloat16),
                                preferred_element_type=jnp.float32)
                        + bc_ref[...])


def _tc_call(x, craw, m_in, m_out, wxt, wht, b4, wm1t, wm2t, bm2, wct, bc2,
             interpret=False):
    full = lambda a: pl.BlockSpec(a.shape, lambda k, nd=a.ndim: (0,) * nd)
    return pl.pallas_call(
        _tc_body,
        grid=(K_TILES,),
        in_specs=[
            full(x),
            full(craw),
            pl.BlockSpec((E_B, N), lambda k: (k, 0)),
            pl.BlockSpec((E_B, N), lambda k: (k, 0)),
            full(wxt), full(wht), full(b4),
            full(wm1t), full(wm2t), full(bm2), full(wct), full(bc2),
        ],
        out_specs=pl.BlockSpec((N, D), lambda k: (0, 0)),
        out_shape=jax.ShapeDtypeStruct((N, D), jnp.float32),
        scratch_shapes=[
            pltpu.VMEM((N, N), jnp.bfloat16),
            pltpu.VMEM((N, D), jnp.bfloat16),
            pltpu.VMEM((N, D), jnp.bfloat16),
            pltpu.VMEM((D, N), jnp.float32),
        ],
        interpret=interpret,
    )(x, craw, m_in, m_out, wxt, wht, b4, wm1t, wm2t, bm2, wct, bc2)


def kernel(x, edge_index, m_in, m_out, Wi, bi, Wf, bf, Wo, bo, Wg, bg, Wm, bm, Wc, bc):
    n = x.shape[1]
    ar = jnp.arange(n, dtype=edge_index.dtype)
    src = jnp.concatenate([edge_index[0], ar]).astype(jnp.int32)
    dst = jnp.concatenate([edge_index[1], ar]).astype(jnp.int32)
    flat = dst * NP + src
    flat_pad = jnp.concatenate(
        [flat, jnp.full((E_PAD - E_REAL,), SENTINEL, jnp.int32)])

    craw = _sc_counts(jnp.reshape(flat_pad, (E_ROWS, 128)))

    wall = jnp.concatenate([Wi, Wf, Wo, Wg], axis=0)
    wxt = wall[:, :D].T
    wht = wall[:, D:].T
    b4 = jnp.concatenate([bi, bf, bo, bg])[None, :]
    wm1t = Wm[:, :D].T
    wm2t = Wm[:, D:].T

    return _tc_call(x, craw, m_in.astype(jnp.bfloat16),
                    m_out.astype(jnp.bfloat16), wxt, wht, b4, wm1t, wm2t,
                    bm[None, :], Wc.T, bc[None, :])


# speedup vs baseline: 1.7898x; 1.7898x over previous
import functools

import jax
import jax.numpy as jnp
from jax import lax
from jax.experimental import pallas as pl
from jax.experimental.pallas import tpu as pltpu
from jax.experimental.pallas import tpu_sc as plsc

N = 1000
NP = 1024
D = 128
HID = 512
T_STEPS = 10
E_REAL = 17000
E_ROWS = 136
E_PAD = E_ROWS * 128
SENTINEL = 1023 * NP + 1023
RPT = NP // 32

E_B = 640
K_TILES = 16000 // E_B


def _sc_counts(flat2d):
    mesh = plsc.VectorSubcoreMesh(core_axis_name="c", subcore_axis_name="s")

    @functools.partial(
        pl.kernel,
        out_type=jax.ShapeDtypeStruct((8 * NP, 128), jnp.float32),
        mesh=mesh,
        compiler_params=pltpu.CompilerParams(needs_layout_passes=False),
        scratch_types=[
            pltpu.VMEM((E_ROWS, 128), jnp.int32),
            pltpu.VMEM((RPT * 8 * 128,), jnp.float32),
            pltpu.VMEM((RPT * 8, 128), jnp.float32),
        ],
    )
    def sc_kernel(flat_hbm, out_hbm, idx_v, cflat_v, c2d_v):
        c = lax.axis_index("c")
        s = lax.axis_index("s")
        w = s * 2 + c
        lo = w * RPT
        pltpu.sync_copy(flat_hbm, idx_v)

        zero16 = jnp.zeros((16,), jnp.float32)
        ones16 = jnp.full((16,), 1.0, jnp.float32)

        def zz(i, _):
            cflat_v[pl.ds(i * 16, 16)] = zero16
            return 0

        lax.fori_loop(0, RPT * 8 * 8, zz, 0)

        def scan(gi, _):
            fl = idx_v[gi // 8, pl.ds((gi % 8) * 16, 16)]
            d = lax.shift_right_logical(fl, 10)
            lane = lax.bitwise_and(fl, 127)
            j = lax.bitwise_and(lax.shift_right_logical(fl, 7), 7)
            dl = d - lo
            m = (dl >= 0) & (dl < RPT)
            tgt = (j * RPT + dl) * 128 + lane
            tgt = jnp.where(m, tgt, 0)
            plsc.addupdate_scatter(cflat_v, [tgt], ones16, mask=m)
            return 0

        lax.fori_loop(0, E_PAD // 16, scan, 0)

        def cp(r, _):
            for g in range(8):
                c2d_v[r, pl.ds(g * 16, 16)] = cflat_v[pl.ds(r * 128 + g * 16, 16)]
            return 0

        lax.fori_loop(0, RPT * 8, cp, 0)

        for j in range(8):
            pltpu.sync_copy(c2d_v.at[pl.ds(j * RPT, RPT)],
                            out_hbm.at[pl.ds(j * NP + w * RPT, RPT)])

    return sc_kernel(flat2d)


def _dot16(a, b):
    return jnp.dot(a.astype(jnp.bfloat16), b.astype(jnp.bfloat16),
                   preferred_element_type=jnp.float32)


def _tc_body(x_ref, craw_ref, min_ref, mout_ref, wxt_ref, wht_ref, b4_ref,
             wm1t_ref, wm2t_ref, bm_ref, wct_ref, bc_ref, out_ref,
             a16_ref, aproj_ref, bproj_ref, xn_ref):
    k = pl.program_id(0)

    @pl.when(k == 0)
    def _prologue():
        cs = jnp.concatenate(
            [craw_ref[j * NP:j * NP + N, :] for j in range(8)], axis=1)[:, :N]
        deg = jnp.sum(cs, axis=1)
        dinv = lax.rsqrt(deg)
        a16_ref[...] = (cs * dinv[:, None] * dinv[None, :]).astype(jnp.bfloat16)

        wx16 = wxt_ref[...].astype(jnp.bfloat16)
        wh16 = wht_ref[...].astype(jnp.bfloat16)

        def step(t, hc):
            h, cc = hc
            xt = x_ref[t].astype(jnp.bfloat16)
            xw = (jnp.dot(xt, wx16, preferred_element_type=jnp.float32)
                  + jnp.dot(h.astype(jnp.bfloat16), wh16,
                            preferred_element_type=jnp.float32))
            z = (jnp.dot(a16_ref[...], xw.astype(jnp.bfloat16),
                         preferred_element_type=jnp.float32)
                 + b4_ref[...])
            ig = jax.nn.sigmoid(z[:, 0:128])
            fg = jax.nn.sigmoid(z[:, 128:256])
            og = jax.nn.sigmoid(z[:, 256:384])
            gg = jnp.tanh(z[:, 384:512])
            cc = fg * cc + ig * gg
            h = og * jnp.tanh(cc)
            return h, cc

        h0 = jnp.zeros((N, D), jnp.float32)
        h, _ = lax.fori_loop(0, T_STEPS, step, (h0, h0))
        aproj_ref[...] = _dot16(h, wm1t_ref[...]).astype(jnp.bfloat16)
        bproj_ref[...] = _dot16(h, wm2t_ref[...]).astype(jnp.bfloat16)
        xn_ref[...] = jnp.zeros((N, D), jnp.float32)

    mit16 = min_ref[...].astype(jnp.bfloat16)
    mot16 = mout_ref[...].astype(jnp.bfloat16)
    et = jnp.maximum(
        lax.dot_general(aproj_ref[...], mit16, (((0,), (0,)), ((), ())),
                        preferred_element_type=jnp.float32)
        + lax.dot_general(bproj_ref[...], mot16, (((0,), (0,)), ((), ())),
                          preferred_element_type=jnp.float32)
        + bm_ref[...], 0.0)
    xn_ref[...] += lax.dot_general(mit16, et.astype(jnp.bfloat16),
                                   (((1,), (1,)), ((), ())),
                                   preferred_element_type=jnp.float32)

    @pl.when(k == K_TILES - 1)
    def _epilogue():
        xw = _dot16(xn_ref[...] * (1.0 / N), wct_ref[...])
        out_ref[...] = (jnp.dot(a16_ref[...], xw.astype(jnp.bfloat16),
                                preferred_element_type=jnp.float32)
                        + bc_ref[...])


def _tc_call(x, craw, m_in, m_out, wxt, wht, b4, wm1t, wm2t, bm2, wct, bc2,
             interpret=False):
    full = lambda a: pl.BlockSpec(a.shape, lambda k, nd=a.ndim: (0,) * nd)
    return pl.pallas_call(
        _tc_body,
        grid=(K_TILES,),
        in_specs=[
            full(x),
            full(craw),
            pl.BlockSpec((N, E_B), lambda k: (0, k)),
            pl.BlockSpec((N, E_B), lambda k: (0, k)),
            full(wxt), full(wht), full(b4),
            full(wm1t), full(wm2t), full(bm2), full(wct), full(bc2),
        ],
        out_specs=pl.BlockSpec((N, D), lambda k: (0, 0)),
        out_shape=jax.ShapeDtypeStruct((N, D), jnp.float32),
        scratch_shapes=[
            pltpu.VMEM((N, N), jnp.bfloat16),
            pltpu.VMEM((N, D), jnp.bfloat16),
            pltpu.VMEM((N, D), jnp.bfloat16),
            pltpu.VMEM((N, D), jnp.float32),
        ],
        interpret=interpret,
    )(x, craw, m_in, m_out, wxt, wht, b4, wm1t, wm2t, bm2, wct, bc2)


def kernel(x, edge_index, m_in, m_out, Wi, bi, Wf, bf, Wo, bo, Wg, bg, Wm, bm, Wc, bc):
    n = x.shape[1]
    ar = jnp.arange(n, dtype=edge_index.dtype)
    src = jnp.concatenate([edge_index[0], ar]).astype(jnp.int32)
    dst = jnp.concatenate([edge_index[1], ar]).astype(jnp.int32)
    flat = dst * NP + src
    flat_pad = jnp.concatenate(
        [flat, jnp.full((E_PAD - E_REAL,), SENTINEL, jnp.int32)])

    craw = _sc_counts(jnp.reshape(flat_pad, (E_ROWS, 128)))

    wall = jnp.concatenate([Wi, Wf, Wo, Wg], axis=0)
    wxt = wall[:, :D].T
    wht = wall[:, D:].T
    b4 = jnp.concatenate([bi, bf, bo, bg])[None, :]
    wm1t = Wm[:, :D].T
    wm2t = Wm[:, D:].T

    return _tc_call(x, craw, m_in.T, m_out.T, wxt, wht, b4, wm1t, wm2t,
                    bm[:, None], Wc.T, bc[None, :])


# speedup vs baseline: 1.8064x; 1.0093x over previous
import functools

import jax
import jax.numpy as jnp
from jax import lax
from jax.experimental import pallas as pl
from jax.experimental.pallas import tpu as pltpu
from jax.experimental.pallas import tpu_sc as plsc

N = 1000
NP = 1024
D = 128
HID = 512
T_STEPS = 10
E_REAL = 17000
E_ROWS = 136
E_PAD = E_ROWS * 128
SENTINEL = 1023 * NP + 1023
RPT = NP // 32

E_B = 640
K_TILES = 16000 // E_B


def _sc_counts(flat2d):
    mesh = plsc.VectorSubcoreMesh(core_axis_name="c", subcore_axis_name="s")

    @functools.partial(
        pl.kernel,
        out_type=jax.ShapeDtypeStruct((8 * NP, 128), jnp.float32),
        mesh=mesh,
        compiler_params=pltpu.CompilerParams(needs_layout_passes=False),
        scratch_types=[
            pltpu.VMEM((E_ROWS, 128), jnp.int32),
            pltpu.VMEM((RPT * 8 * 128,), jnp.float32),
            pltpu.VMEM((RPT * 8, 128), jnp.float32),
        ],
    )
    def sc_kernel(flat_hbm, out_hbm, idx_v, cflat_v, c2d_v):
        c = lax.axis_index("c")
        s = lax.axis_index("s")
        w = s * 2 + c
        lo = w * RPT
        pltpu.sync_copy(flat_hbm, idx_v)

        zero16 = jnp.zeros((16,), jnp.float32)
        ones16 = jnp.full((16,), 1.0, jnp.float32)

        def zz(i, _):
            cflat_v[pl.ds(i * 16, 16)] = zero16
            return 0

        lax.fori_loop(0, RPT * 8 * 8, zz, 0)

        def scan(r, _):
            for g in range(8):
                fl = idx_v[r, pl.ds(g * 16, 16)]
                d = lax.shift_right_logical(fl, 10)
                lane = lax.bitwise_and(fl, 127)
                j = lax.bitwise_and(lax.shift_right_logical(fl, 7), 7)
                dl = d - lo
                m = (dl >= 0) & (dl < RPT)
                tgt = (j * RPT + dl) * 128 + lane
                tgt = jnp.where(m, tgt, 0)
                plsc.addupdate_scatter(cflat_v, [tgt], ones16, mask=m)
            return 0

        lax.fori_loop(0, E_ROWS, scan, 0)

        def cp(r, _):
            for g in range(8):
                c2d_v[r, pl.ds(g * 16, 16)] = cflat_v[pl.ds(r * 128 + g * 16, 16)]
            return 0

        lax.fori_loop(0, RPT * 8, cp, 0)

        for j in range(8):
            pltpu.sync_copy(c2d_v.at[pl.ds(j * RPT, RPT)],
                            out_hbm.at[pl.ds(j * NP + w * RPT, RPT)])

    return sc_kernel(flat2d)


def _dot16(a, b):
    return jnp.dot(a.astype(jnp.bfloat16), b.astype(jnp.bfloat16),
                   preferred_element_type=jnp.float32)


def _tc_body(x_ref, craw_ref, min_ref, mout_ref, wxt_ref, wht_ref, b4_ref,
             wm1t_ref, wm2t_ref, bm_ref, wct_ref, bc_ref, out_ref,
             a16_ref, aproj_ref, bproj_ref, xn_ref):
    k = pl.program_id(0)

    @pl.when(k == 0)
    def _prologue():
        cs = jnp.concatenate(
            [craw_ref[j * NP:j * NP + N, :] for j in range(8)], axis=1)[:, :N]
        deg = jnp.sum(cs, axis=1)
        dinv = lax.rsqrt(deg)
        a16_ref[...] = (cs * dinv[:, None] * dinv[None, :]).astype(jnp.bfloat16)

        wx16 = wxt_ref[...].astype(jnp.bfloat16)
        wh16 = wht_ref[...].astype(jnp.bfloat16)

        def step(t, hc):
            h, cc = hc
            xt = x_ref[t].astype(jnp.bfloat16)
            xw = (jnp.dot(xt, wx16, preferred_element_type=jnp.float32)
                  + jnp.dot(h.astype(jnp.bfloat16), wh16,
                            preferred_element_type=jnp.float32))
            z = (jnp.dot(a16_ref[...], xw.astype(jnp.bfloat16),
                         preferred_element_type=jnp.float32)
                 + b4_ref[...])
            ig = jax.nn.sigmoid(z[:, 0:128])
            fg = jax.nn.sigmoid(z[:, 128:256])
            og = jax.nn.sigmoid(z[:, 256:384])
            gg = jnp.tanh(z[:, 384:512])
            cc = fg * cc + ig * gg
            h = og * jnp.tanh(cc)
            return h, cc

        h0 = jnp.zeros((N, D), jnp.float32)
        h, _ = lax.fori_loop(0, T_STEPS, step, (h0, h0))
        aproj_ref[...] = _dot16(h, wm1t_ref[...]).astype(jnp.bfloat16)
        bproj_ref[...] = _dot16(h, wm2t_ref[...]).astype(jnp.bfloat16)
        xn_ref[...] = jnp.zeros((N, D), jnp.float32)

    mit16 = min_ref[...].astype(jnp.bfloat16)
    mot16 = mout_ref[...].astype(jnp.bfloat16)
    et = jnp.maximum(
        lax.dot_general(aproj_ref[...], mit16, (((0,), (0,)), ((), ())),
                        preferred_element_type=jnp.float32)
        + lax.dot_general(bproj_ref[...], mot16, (((0,), (0,)), ((), ())),
                          preferred_element_type=jnp.float32)
        + bm_ref[...], 0.0)
    xn_ref[...] += lax.dot_general(mit16, et.astype(jnp.bfloat16),
                                   (((1,), (1,)), ((), ())),
                                   preferred_element_type=jnp.float32)

    @pl.when(k == K_TILES - 1)
    def _epilogue():
        xw = _dot16(xn_ref[...] * (1.0 / N), wct_ref[...])
        out_ref[...] = (jnp.dot(a16_ref[...], xw.astype(jnp.bfloat16),
                                preferred_element_type=jnp.float32)
                        + bc_ref[...])


def _tc_call(x, craw, m_in, m_out, wxt, wht, b4, wm1t, wm2t, bm2, wct, bc2,
             interpret=False):
    full = lambda a: pl.BlockSpec(a.shape, lambda k, nd=a.ndim: (0,) * nd)
    return pl.pallas_call(
        _tc_body,
        grid=(K_TILES,),
        in_specs=[
            full(x),
            full(craw),
            pl.BlockSpec((N, E_B), lambda k: (0, k)),
            pl.BlockSpec((N, E_B), lambda k: (0, k)),
            full(wxt), full(wht), full(b4),
            full(wm1t), full(wm2t), full(bm2), full(wct), full(bc2),
        ],
        out_specs=pl.BlockSpec((N, D), lambda k: (0, 0)),
        out_shape=jax.ShapeDtypeStruct((N, D), jnp.float32),
        scratch_shapes=[
            pltpu.VMEM((N, N), jnp.bfloat16),
            pltpu.VMEM((N, D), jnp.bfloat16),
            pltpu.VMEM((N, D), jnp.bfloat16),
            pltpu.VMEM((N, D), jnp.float32),
        ],
        interpret=interpret,
    )(x, craw, m_in, m_out, wxt, wht, b4, wm1t, wm2t, bm2, wct, bc2)


def kernel(x, edge_index, m_in, m_out, Wi, bi, Wf, bf, Wo, bo, Wg, bg, Wm, bm, Wc, bc):
    n = x.shape[1]
    ar = jnp.arange(n, dtype=edge_index.dtype)
    src = jnp.concatenate([edge_index[0], ar]).astype(jnp.int32)
    dst = jnp.concatenate([edge_index[1], ar]).astype(jnp.int32)
    flat = dst * NP + src
    flat_pad = jnp.concatenate(
        [flat, jnp.full((E_PAD - E_REAL,), SENTINEL, jnp.int32)])

    craw = _sc_counts(jnp.reshape(flat_pad, (E_ROWS, 128)))

    wall = jnp.concatenate([Wi, Wf, Wo, Wg], axis=0)
    wxt = wall[:, :D].T
    wht = wall[:, D:].T
    b4 = jnp.concatenate([bi, bf, bo, bg])[None, :]
    wm1t = Wm[:, :D].T
    wm2t = Wm[:, D:].T

    return _tc_call(x, craw, m_in.T, m_out.T, wxt, wht, b4, wm1t, wm2t,
                    bm[:, None], Wc.T, bc[None, :])


# speedup vs baseline: 1.8079x; 1.0008x over previous
import functools

import jax
import jax.numpy as jnp
from jax import lax
from jax.experimental import pallas as pl
from jax.experimental.pallas import tpu as pltpu
from jax.experimental.pallas import tpu_sc as plsc

N = 1000
NP = 1024
D = 128
HID = 512
T_STEPS = 10
E_REAL = 17000
E_ROWS = 136
E_PAD = E_ROWS * 128
SENTINEL = 1023 * NP + 1023
RPT = NP // 32

E_B = 640
K_TILES = 16000 // E_B


def _sc_counts(flat2d):
    mesh = plsc.VectorSubcoreMesh(core_axis_name="c", subcore_axis_name="s")

    @functools.partial(
        pl.kernel,
        out_type=jax.ShapeDtypeStruct((8 * NP, 128), jnp.float32),
        mesh=mesh,
        compiler_params=pltpu.CompilerParams(needs_layout_passes=False),
        scratch_types=[
            pltpu.VMEM((E_ROWS, 128), jnp.int32),
            pltpu.VMEM((RPT * 8 * 128,), jnp.float32),
            pltpu.VMEM((RPT * 8, 128), jnp.float32),
        ],
    )
    def sc_kernel(flat_hbm, out_hbm, idx_v, cflat_v, c2d_v):
        c = lax.axis_index("c")
        s = lax.axis_index("s")
        w = s * 2 + c
        lo = w * RPT
        pltpu.sync_copy(flat_hbm, idx_v)

        zero16 = jnp.zeros((16,), jnp.float32)
        ones16 = jnp.full((16,), 1.0, jnp.float32)

        def zz(i, _):
            cflat_v[pl.ds(i * 16, 16)] = zero16
            return 0

        lax.fori_loop(0, RPT * 8 * 8, zz, 0)

        def scan(r, _):
            for g in range(8):
                fl = idx_v[r, pl.ds(g * 16, 16)]
                d = lax.shift_right_logical(fl, 10)
                lane = lax.bitwise_and(fl, 127)
                j = lax.bitwise_and(lax.shift_right_logical(fl, 7), 7)
                dl = d - lo
                m = (dl >= 0) & (dl < RPT)
                tgt = (j * RPT + dl) * 128 + lane
                tgt = jnp.where(m, tgt, 0)
                plsc.addupdate_scatter(cflat_v, [tgt], ones16, mask=m)
            return 0

        lax.fori_loop(0, E_ROWS, scan, 0)

        def cp(r, _):
            for g in range(8):
                c2d_v[r, pl.ds(g * 16, 16)] = cflat_v[pl.ds(r * 128 + g * 16, 16)]
            return 0

        lax.fori_loop(0, RPT * 8, cp, 0)

        for j in range(8):
            pltpu.sync_copy(c2d_v.at[pl.ds(j * RPT, RPT)],
                            out_hbm.at[pl.ds(j * NP + w * RPT, RPT)])

    return sc_kernel(flat2d)


def _dot16(a, b):
    return jnp.dot(a.astype(jnp.bfloat16), b.astype(jnp.bfloat16),
                   preferred_element_type=jnp.float32)


def _tc_body(x_ref, craw_ref, min_ref, mout_ref, wxt_ref, wht_ref, b4_ref,
             wm1t_ref, wm2t_ref, bm_ref, wct_ref, bc_ref, out_ref,
             a16_ref, aproj_ref, bproj_ref, xn_ref):
    k = pl.program_id(0)

    @pl.when(k == 0)
    def _prologue():
        cs = jnp.concatenate(
            [craw_ref[j * NP:j * NP + N, :] for j in range(8)], axis=1)[:, :N]
        deg = jnp.sum(cs, axis=1)
        dinv = lax.rsqrt(deg)
        a16_ref[...] = (cs * dinv[:, None] * dinv[None, :]).astype(jnp.bfloat16)

        wx16 = wxt_ref[...].astype(jnp.bfloat16)
        wh16 = wht_ref[...].astype(jnp.bfloat16)

        def step(t, hc):
            h, cc = hc
            xt = x_ref[t].astype(jnp.bfloat16)
            xw = (jnp.dot(xt, wx16, preferred_element_type=jnp.float32)
                  + jnp.dot(h.astype(jnp.bfloat16), wh16,
                            preferred_element_type=jnp.float32))
            z = (jnp.dot(a16_ref[...], xw.astype(jnp.bfloat16),
                         preferred_element_type=jnp.float32)
                 + b4_ref[...])
            ig = jax.nn.sigmoid(z[:, 0:128])
            fg = jax.nn.sigmoid(z[:, 128:256])
            og = jax.nn.sigmoid(z[:, 256:384])
            gg = jnp.tanh(z[:, 384:512])
            cc = fg * cc + ig * gg
            h = og * jnp.tanh(cc)
            return h, cc

        h0 = jnp.zeros((N, D), jnp.float32)
        h, _ = lax.fori_loop(0, T_STEPS, step, (h0, h0))
        aproj_ref[...] = _dot16(h, wm1t_ref[...]).astype(jnp.bfloat16)
        bproj_ref[...] = _dot16(h, wm2t_ref[...]).astype(jnp.bfloat16)
        xn_ref[...] = jnp.zeros((N, D), jnp.float32)

    mit16 = min_ref[...].astype(jnp.bfloat16)
    mot16 = mout_ref[...].astype(jnp.bfloat16)
    et = jnp.maximum(
        lax.dot_general(aproj_ref[...], mit16, (((0,), (0,)), ((), ())),
                        preferred_element_type=jnp.float32)
        + lax.dot_general(bproj_ref[...], mot16, (((0,), (0,)), ((), ())),
                          preferred_element_type=jnp.float32)
        + bm_ref[...], 0.0)
    xn_ref[...] += lax.dot_general(mit16, et.astype(jnp.bfloat16),
                                   (((1,), (1,)), ((), ())),
                                   preferred_element_type=jnp.float32)

    @pl.when(k == K_TILES - 1)
    def _epilogue():
        xw = _dot16(xn_ref[...] * (1.0 / N), wct_ref[...])
        out_ref[...] = (jnp.dot(a16_ref[...], xw.astype(jnp.bfloat16),
                                preferred_element_type=jnp.float32)
                        + bc_ref[...])


def _tc_call(x, craw, m_in, m_out, wxt, wht, b4, wm1t, wm2t, bm2, wct, bc2):
    full = lambda a: pl.BlockSpec(a.shape, lambda k, nd=a.ndim: (0,) * nd)
    return pl.pallas_call(
        _tc_body,
        grid=(K_TILES,),
        in_specs=[
            full(x),
            full(craw),
            pl.BlockSpec((N, E_B), lambda k: (0, k)),
            pl.BlockSpec((N, E_B), lambda k: (0, k)),
            full(wxt), full(wht), full(b4),
            full(wm1t), full(wm2t), full(bm2), full(wct), full(bc2),
        ],
        out_specs=pl.BlockSpec((N, D), lambda k: (0, 0)),
        out_shape=jax.ShapeDtypeStruct((N, D), jnp.float32),
        scratch_shapes=[
            pltpu.VMEM((N, N), jnp.bfloat16),
            pltpu.VMEM((N, D), jnp.bfloat16),
            pltpu.VMEM((N, D), jnp.bfloat16),
            pltpu.VMEM((N, D), jnp.float32),
        ],
    )(x, craw, m_in, m_out, wxt, wht, b4, wm1t, wm2t, bm2, wct, bc2)


def kernel(x, edge_index, m_in, m_out, Wi, bi, Wf, bf, Wo, bo, Wg, bg, Wm, bm, Wc, bc):
    n = x.shape[1]
    ar = jnp.arange(n, dtype=edge_index.dtype)
    src = jnp.concatenate([edge_index[0], ar]).astype(jnp.int32)
    dst = jnp.concatenate([edge_index[1], ar]).astype(jnp.int32)
    flat = dst * NP + src
    flat_pad = jnp.concatenate(
        [flat, jnp.full((E_PAD - E_REAL,), SENTINEL, jnp.int32)])

    craw = _sc_counts(jnp.reshape(flat_pad, (E_ROWS, 128)))

    wall = jnp.concatenate([Wi, Wf, Wo, Wg], axis=0)
    wxt = wall[:, :D].T
    wht = wall[:, D:].T
    b4 = jnp.concatenate([bi, bf, bo, bg])[None, :]
    wm1t = Wm[:, :D].T
    wm2t = Wm[:, D:].T

    return _tc_call(x, craw, m_in.T, m_out.T, wxt, wht, b4, wm1t, wm2t,
                    bm[:, None], Wc.T, bc[None, :])
